# Initial kernel scaffold; baseline (speedup 1.0000x reference)
import jax
import jax.numpy as jnp
from jax.experimental import pallas as pl


def kernel(x, edge_index, edge_weight, batch, W1, W2, Wp, bp, gamma, beta, alpha):
    raise NotImplementedError("write your pallas kernel here")


# speedup vs baseline: 4.0791x; 4.0791x over previous
import functools

import jax
import jax.numpy as jnp
from jax import lax
from jax.experimental import pallas as pl
from jax.experimental.pallas import tpu as pltpu
from jax.experimental.pallas import tpu_sc as plsc

N = 10000
E = 320000
D = 128
G = 512

NC = 2
NS = 16
NW = NC * NS

NPAD = 10240
ROWS_PT = NPAD // NS
RCH = 128
NRC = ROWS_PT // RCH

ECH = 128
EPT_CH = 79
EPT = ECH * EPT_CH
EPAD = EPT * NW
POOL_PT = G // NS


def _sc_body(x_hbm, src_hbm, dst_hbm, ew_hbm, batch_hbm,
             agg_out, pool_out,
             agg_sh, pool_sh, zbuf, src_v, dst_v, ew_v, rows_v, bidx_v, sem):
    c = lax.axis_index("c")
    s = lax.axis_index("s")
    wid = s * NC + c
    r0 = s * ROWS_PT

    zvec = jnp.zeros((16,), jnp.float32)

    def _zrow(i, carry):
        for j in range(D // 16):
            zbuf[i, pl.ds(j * 16, 16)] = zvec
        return carry

    lax.fori_loop(0, RCH, _zrow, 0)

    @pl.when(c == 0)
    def _():
        for k in range(NRC):
            off = r0 + k * RCH
            pltpu.sync_copy(x_hbm.at[pl.ds(off, RCH)],
                            agg_sh.at[pl.ds(off, RCH)])

    @pl.when(c != 0)
    def _():
        for k in range(NRC):
            off = r0 + k * RCH
            pltpu.sync_copy(zbuf, agg_sh.at[pl.ds(off, RCH)])

    pltpu.sync_copy(zbuf.at[pl.ds(0, POOL_PT)],
                    pool_sh.at[pl.ds(s * POOL_PT, POOL_PT)])
    plsc.subcore_barrier()

    ebase = wid * EPT

    def _echunk(ci, carry):
        e = ebase + ci * ECH
        pltpu.sync_copy(src_hbm.at[pl.ds(e, ECH)], src_v)
        pltpu.sync_copy(dst_hbm.at[pl.ds(e, ECH)], dst_v)
        pltpu.sync_copy(ew_hbm.at[pl.ds(e, ECH)], ew_v)
        pltpu.async_copy(x_hbm.at[src_v], rows_v, sem).wait()

        def _scale(g, inner):
            wv = ew_v[pl.ds(g * 16, 16)]
            for e in range(16):
                w = wv[e]
                r = g * 16 + e
                for j in range(D // 16):
                    sl = pl.ds(j * 16, 16)
                    rows_v[r, sl] = rows_v[r, sl] * w
            return inner

        lax.fori_loop(0, ECH // 16, _scale, 0)
        pltpu.sync_copy(rows_v, agg_sh.at[dst_v], add=True)
        return carry

    lax.fori_loop(0, EPT_CH, _echunk, 0)
    plsc.subcore_barrier()

    for k in range(NRC):
        off = r0 + k * RCH
        pltpu.sync_copy(agg_sh.at[pl.ds(off, RCH)], rows_v)
        pltpu.sync_copy(batch_hbm.at[pl.ds(off, RCH)], bidx_v)
        pltpu.sync_copy(rows_v, agg_out.at[pl.ds(c * NPAD + off, RCH)])
        pltpu.sync_copy(rows_v, pool_sh.at[bidx_v], add=True)
    plsc.subcore_barrier()

    pltpu.sync_copy(pool_sh.at[pl.ds(s * POOL_PT, POOL_PT)],
                    pool_out.at[pl.ds(c * G + s * POOL_PT, POOL_PT)])


_sc_call = pl.kernel(
    _sc_body,
    out_type=[
        jax.ShapeDtypeStruct((NC * NPAD, D), jnp.float32),
        jax.ShapeDtypeStruct((NC * G, D), jnp.float32),
    ],
    mesh=plsc.VectorSubcoreMesh(core_axis_name="c", subcore_axis_name="s"),
    scratch_types=[
        pltpu.VMEM_SHARED((NPAD, D), jnp.float32),
        pltpu.VMEM_SHARED((G, D), jnp.float32),
        pltpu.VMEM((RCH, D), jnp.float32),
        pltpu.VMEM((ECH,), jnp.int32),
        pltpu.VMEM((ECH,), jnp.int32),
        pltpu.VMEM((ECH,), jnp.float32),
        pltpu.VMEM((max(ECH, RCH), D), jnp.float32),
        pltpu.VMEM((RCH,), jnp.int32),
        pltpu.SemaphoreType.DMA,
    ],
)


def _small_body(p0_ref, p1_ref, w1_ref, w2_ref, wp_ref,
                g1_ref, gt_ref, wc_ref):
    s = p0_ref[...] + p1_ref[...]
    g1_ref[...] = jnp.dot(s, w1_ref[...], preferred_element_type=jnp.float32)
    gt_ref[...] = jnp.dot(s, w2_ref[...], preferred_element_type=jnp.float32)
    wc_ref[...] = jnp.dot(w2_ref[...], wp_ref[...],
                          preferred_element_type=jnp.float32)


def _pred_body(a0_ref, a1_ref, wc_ref, pv_ref, out_ref):
    a = a0_ref[...] + a1_ref[...]
    z = jnp.dot(a, wc_ref[...], preferred_element_type=jnp.float32)
    z = z + pv_ref[0:1, :]
    mu = jnp.mean(z, axis=-1, keepdims=True)
    zc = z - mu
    var = jnp.mean(zc * zc, axis=-1, keepdims=True)
    zn = zc * lax.rsqrt(var + 1e-5) * pv_ref[1:2, :] + pv_ref[2:3, :]
    alpha = pv_ref[3, 0]
    out_ref[...] = jnp.where(zn >= 0, zn, alpha * zn)


def kernel(x, edge_index, edge_weight, batch, W1, W2, Wp, bp, gamma, beta, alpha):
    x_pad = jnp.pad(x, ((0, NPAD - N), (0, 0)))
    batch_pad = jnp.pad(batch, (0, NPAD - N))
    src = jnp.pad(edge_index[0], (0, EPAD - E))
    dst = jnp.pad(edge_index[1], (0, EPAD - E))
    ew = jnp.pad(edge_weight, (0, EPAD - E))

    agg_parts, pool_parts = _sc_call(x_pad, src, dst, ew, batch_pad)

    g1, gt, wc = pl.pallas_call(
        _small_body,
        out_shape=[
            jax.ShapeDtypeStruct((G, D), jnp.float32),
            jax.ShapeDtypeStruct((G, D), jnp.float32),
            jax.ShapeDtypeStruct((D, D), jnp.float32),
        ],
    )(pool_parts[:G], pool_parts[G:], W1, W2, Wp)

    pvec = jnp.stack([bp, gamma, beta,
                      jnp.full((D,), alpha, dtype=jnp.float32)] + [bp] * 4)

    nb = 8
    blk = NPAD // nb
    h_full = pl.pallas_call(
        _pred_body,
        grid=(nb,),
        in_specs=[
            pl.BlockSpec((blk, D), lambda i: (i, 0)),
            pl.BlockSpec((blk, D), lambda i: (i, 0)),
            pl.BlockSpec((D, D), lambda i: (0, 0)),
            pl.BlockSpec((8, D), lambda i: (0, 0)),
        ],
        out_specs=pl.BlockSpec((blk, D), lambda i: (i, 0)),
        out_shape=jax.ShapeDtypeStruct((NPAD, D), jnp.float32),
    )(agg_parts[:NPAD], agg_parts[NPAD:], wc, pvec)

    h_pred = h_full[:N]
    return (g1, g1, h_pred, h_pred, gt, gt)
